# baseline (speedup 1.0000x reference)
import functools

import jax
import jax.numpy as jnp
from jax import lax
from jax.experimental import pallas as pl
from jax.experimental.pallas import tpu as pltpu
from jax.experimental.pallas import tpu_sc as plsc

_D = 32
_HIST = 200
_HPAD = 208
_BATCH = 4096
_NW = 32
_RPW = _BATCH // _NW
_G0 = 128
_G1 = _HIST - _G0

_mesh = plsc.VectorSubcoreMesh(core_axis_name="c", subcore_axis_name="s")


@functools.partial(
    pl.kernel,
    out_type=jax.ShapeDtypeStruct((_BATCH, _D), jnp.float32),
    mesh=_mesh,
    scratch_types=[
        pltpu.VMEM((_RPW, _HPAD), jnp.int32),
        pltpu.VMEM((2, _HPAD, _D), jnp.float32),
        pltpu.VMEM((2, _HPAD + 8), jnp.float32),
        pltpu.VMEM((_RPW, _D), jnp.float32),
        pltpu.SemaphoreType.DMA,
        pltpu.SemaphoreType.DMA,
    ],
    compiler_params=pltpu.CompilerParams(use_tc_tiling_on_sc=False),
)
def _nbow(idx_hbm, emb_hbm, tw_hbm, out_hbm, idx_v, ebuf, tbuf, out_v,
          sem0, sem1):
    wid = lax.axis_index("c") * 16 + lax.axis_index("s")
    base = wid * _RPW
    pltpu.sync_copy(idx_hbm.at[pl.ds(base, _RPW)],
                    idx_v.at[:, pl.ds(0, _HIST)])

    z = jnp.zeros((16,), jnp.float32)
    for b in range(2):
        tbuf[b, pl.ds(_HIST, 16)] = z
        for rp in range(_HIST, _HPAD):
            ebuf[b, rp, pl.ds(0, 16)] = z
            ebuf[b, rp, pl.ds(16, 16)] = z

    sems = (sem0, sem1)

    def row_copies(r, b):
        sem = sems[b]
        i0 = idx_v.at[r, pl.ds(0, _G0)]
        i1 = idx_v.at[r, pl.ds(_G0, _G1)]
        return (
            pltpu.make_async_copy(emb_hbm.at[i0], ebuf.at[b, pl.ds(0, _G0)], sem),
            pltpu.make_async_copy(emb_hbm.at[i1], ebuf.at[b, pl.ds(_G0, _G1)], sem),
            pltpu.make_async_copy(tw_hbm.at[i0], tbuf.at[b, pl.ds(0, _G0)], sem),
            pltpu.make_async_copy(tw_hbm.at[i1], tbuf.at[b, pl.ds(_G0, _G1)], sem),
        )

    def start_row(r, b):
        for cp in row_copies(r, b):
            cp.start()

    def wait_row(r, b):
        for cp in row_copies(r, b):
            cp.wait()

    def compute_row(r, b):
        a0 = jnp.zeros((16,), jnp.float32)
        a1 = jnp.zeros((16,), jnp.float32)
        for g in range(_HPAD // 16):
            off = g * 16
            iv = idx_v[r, pl.ds(off, 16)]
            tw16 = tbuf[b, pl.ds(off, 16)]
            tw16 = tw16 * jnp.minimum(iv, 1).astype(jnp.float32)
            for l in range(16):
                w = tw16[l]
                e0 = ebuf[b, off + l, pl.ds(0, 16)]
                e1 = ebuf[b, off + l, pl.ds(16, 16)]
                a0 = a0 + w * e0
                a1 = a1 + w * e1
        out_v[r, pl.ds(0, 16)] = a0
        out_v[r, pl.ds(16, 16)] = a1

    start_row(0, 0)
    start_row(1, 1)

    def pair(p, carry):
        rr = p * 2
        for b in range(2):
            r = rr + b
            wait_row(r, b)
            compute_row(r, b)

            @pl.when(r + 2 < _RPW)
            def _():
                start_row(r + 2, b)
        return carry

    lax.fori_loop(0, _RPW // 2, pair, 0)
    pltpu.sync_copy(out_v, out_hbm.at[pl.ds(base, _RPW)])


_VOCAB = 1_000_000
_CV = 1024
_NFULL = _VOCAB // _CV
_CR = _CV // 4


@functools.partial(
    pl.kernel,
    out_type=jax.ShapeDtypeStruct((_VOCAB // 4, 4 * _D), jnp.float32),
    mesh=_mesh,
    scratch_types=[
        pltpu.VMEM((_D, _CV), jnp.float32),
        pltpu.VMEM((_CR, 4 * _D), jnp.float32),
    ],
    compiler_params=pltpu.CompilerParams(use_tc_tiling_on_sc=True,
                                         needs_layout_passes=False),
)
def _sc_transpose(embt_hbm, tail_hbm, out_hbm, xbuf, obuf):
    w = lax.axis_index("c") * 16 + lax.axis_index("s")
    lane = plsc.cumsum(jnp.ones((16,), jnp.int32)) - 1
    lane4 = lax.shift_right_logical(lane, 2)
    cpat = (lane & 3) * _D

    def chunk_compute():
        def d_body(d, carry):
            col_vec = cpat + d
            for g in range(_CV // 16):
                row_vec = lane4 + 4 * g
                plsc.store_scatter(obuf, [row_vec, col_vec],
                                   xbuf[d, pl.ds(16 * g, 16)])
            return carry
        lax.fori_loop(0, _D, d_body, 0)

    def loop(t, carry):
        c = w + 32 * t

        @pl.when(c < _NFULL)
        def _():
            v0 = c * _CV
            pltpu.sync_copy(embt_hbm.at[:, pl.ds(v0, _CV)], xbuf)
            chunk_compute()
            pltpu.sync_copy(obuf, out_hbm.at[pl.ds(c * _CR, _CR)])

        @pl.when(c == _NFULL)
        def _():
            pltpu.sync_copy(tail_hbm,
                            out_hbm.at[pl.ds(_NFULL * _CR, 144)])

        return carry

    lax.fori_loop(0, (_NFULL + 1 + 31) // 32, loop, 0)


def kernel(idxs, embedding, token_weights):
    tail = embedding[_NFULL * _CV:, :].reshape(144, 4 * _D)
    emb_lin = _sc_transpose(embedding.T, tail)
    emb_lin = emb_lin.reshape(-1).reshape(embedding.shape)
    return _nbow(idxs, emb_lin, token_weights)


# speedup vs baseline: 1.6793x; 1.6793x over previous
import functools

import jax
import jax.numpy as jnp
from jax import lax
from jax.experimental import pallas as pl
from jax.experimental.pallas import tpu as pltpu
from jax.experimental.pallas import tpu_sc as plsc

_D = 32
_HIST = 200
_HPAD = 208
_BATCH = 4096
_NW = 32
_RPW = _BATCH // _NW
_G0 = 128
_G1 = _HIST - _G0

_mesh = plsc.VectorSubcoreMesh(core_axis_name="c", subcore_axis_name="s")


@functools.partial(
    pl.kernel,
    out_type=jax.ShapeDtypeStruct((_BATCH, _D), jnp.float32),
    mesh=_mesh,
    scratch_types=[
        pltpu.VMEM((_RPW, _HPAD), jnp.int32),
        pltpu.VMEM((2, _HPAD, _D), jnp.float32),
        pltpu.VMEM((2, _HPAD + 8), jnp.float32),
        pltpu.VMEM((_RPW, _D), jnp.float32),
        pltpu.SemaphoreType.DMA,
        pltpu.SemaphoreType.DMA,
    ],
    compiler_params=pltpu.CompilerParams(use_tc_tiling_on_sc=False),
)
def _nbow(idx_hbm, emb_hbm, tw_hbm, out_hbm, idx_v, ebuf, tbuf, out_v,
          sem0, sem1):
    wid = lax.axis_index("c") * 16 + lax.axis_index("s")
    base = wid * _RPW
    pltpu.sync_copy(idx_hbm.at[pl.ds(base, _RPW)],
                    idx_v.at[:, pl.ds(0, _HIST)])

    z = jnp.zeros((16,), jnp.float32)
    for b in range(2):
        tbuf[b, pl.ds(_HIST, 16)] = z
        for rp in range(_HIST, _HPAD):
            ebuf[b, rp, pl.ds(0, 16)] = z
            ebuf[b, rp, pl.ds(16, 16)] = z

    sems = (sem0, sem1)

    def row_copies(r, b):
        sem = sems[b]
        i0 = idx_v.at[r, pl.ds(0, _G0)]
        i1 = idx_v.at[r, pl.ds(_G0, _G1)]
        return (
            pltpu.make_async_copy(emb_hbm.at[i0], ebuf.at[b, pl.ds(0, _G0)], sem),
            pltpu.make_async_copy(emb_hbm.at[i1], ebuf.at[b, pl.ds(_G0, _G1)], sem),
            pltpu.make_async_copy(tw_hbm.at[i0], tbuf.at[b, pl.ds(0, _G0)], sem),
            pltpu.make_async_copy(tw_hbm.at[i1], tbuf.at[b, pl.ds(_G0, _G1)], sem),
        )

    def start_row(r, b):
        for cp in row_copies(r, b):
            cp.start()

    def wait_row(r, b):
        for cp in row_copies(r, b):
            cp.wait()

    def compute_row(r, b):
        a0 = jnp.zeros((16,), jnp.float32)
        a1 = jnp.zeros((16,), jnp.float32)
        for g in range(_HPAD // 16):
            off = g * 16
            iv = idx_v[r, pl.ds(off, 16)]
            tw16 = tbuf[b, pl.ds(off, 16)]
            tw16 = tw16 * jnp.minimum(iv, 1).astype(jnp.float32)
            for l in range(16):
                w = tw16[l]
                e0 = ebuf[b, off + l, pl.ds(0, 16)]
                e1 = ebuf[b, off + l, pl.ds(16, 16)]
                a0 = a0 + w * e0
                a1 = a1 + w * e1
        out_v[r, pl.ds(0, 16)] = a0
        out_v[r, pl.ds(16, 16)] = a1

    start_row(0, 0)
    start_row(1, 1)

    def pair(p, carry):
        rr = p * 2
        for b in range(2):
            r = rr + b
            wait_row(r, b)
            compute_row(r, b)

            @pl.when(r + 2 < _RPW)
            def _():
                start_row(r + 2, b)
        return carry

    lax.fori_loop(0, _RPW // 2, pair, 0)
    pltpu.sync_copy(out_v, out_hbm.at[pl.ds(base, _RPW)])


_VB = 32768
_VOCAB = 1_000_000
_TGRID = (_VOCAB + _VB - 1) // _VB


def _tp_body(x_ref, o_ref):
    xt = x_ref[...].T.reshape(_VB // 4, 4, _D)
    o_ref[...] = jnp.concatenate([xt[:, q, :] for q in range(4)], axis=1)


def _tc_transpose(emb_t):
    return pl.pallas_call(
        _tp_body,
        grid=(_TGRID,),
        in_specs=[pl.BlockSpec((_D, _VB), lambda i: (0, i))],
        out_specs=pl.BlockSpec((_VB // 4, 4 * _D), lambda i: (i, 0)),
        out_shape=jax.ShapeDtypeStruct((_VOCAB // 4, 4 * _D), jnp.float32),
    )(emb_t)


def kernel(idxs, embedding, token_weights):
    emb_lin = _tc_transpose(embedding.T)
    emb_lin = emb_lin.reshape(-1).reshape(embedding.shape)
    return _nbow(idxs, emb_lin, token_weights)
